# Initial kernel scaffold; baseline (speedup 1.0000x reference)
import jax
import jax.numpy as jnp
from jax.experimental import pallas as pl


def kernel(x, W, b):
    raise NotImplementedError("write your pallas kernel here")


# speedup vs baseline: 2.5939x; 2.5939x over previous
import jax
import jax.numpy as jnp
from jax.experimental import pallas as pl

_BLK = 512


def _fused_kernel(x_ref, w_ref, b_ref, out_ref):
    xb = x_ref[...]
    nan = jnp.isnan(xb)
    input_pred = jnp.where(nan, 0.0, xb)
    pred = jnp.sum(input_pred * w_ref[...], axis=1, keepdims=True) + b_ref[0, 0]
    d = xb.shape[1]
    col = jax.lax.broadcasted_iota(jnp.int32, xb.shape, 1)
    last_fixed = jnp.where(nan, pred, xb)
    out_ref[...] = jnp.where(col == d - 1, last_fixed, input_pred)


def kernel(x, W, b):
    n, d = x.shape
    w2 = W.reshape(1, d)
    b2 = b.reshape(1, 1)
    grid = (n // _BLK,)
    return pl.pallas_call(
        _fused_kernel,
        grid=grid,
        in_specs=[
            pl.BlockSpec((_BLK, d), lambda i: (i, 0)),
            pl.BlockSpec((1, d), lambda i: (0, 0)),
            pl.BlockSpec((1, 1), lambda i: (0, 0)),
        ],
        out_specs=pl.BlockSpec((_BLK, d), lambda i: (i, 0)),
        out_shape=jax.ShapeDtypeStruct((n, d), x.dtype),
    )(x, w2, b2)


# speedup vs baseline: 2.6600x; 1.0255x over previous
import jax
import jax.numpy as jnp
from jax.experimental import pallas as pl

_BLK = 1024


def _fused_kernel(x_ref, w_ref, b_ref, out_ref):
    xb = x_ref[...]
    nan = jnp.isnan(xb)
    input_pred = jnp.where(nan, 0.0, xb)
    pred = jnp.sum(input_pred * w_ref[...], axis=1, keepdims=True) + b_ref[0, 0]
    d = xb.shape[1]
    col = jax.lax.broadcasted_iota(jnp.int32, xb.shape, 1)
    last_fixed = jnp.where(nan, pred, xb)
    out_ref[...] = jnp.where(col == d - 1, last_fixed, input_pred)


def kernel(x, W, b):
    n, d = x.shape
    w2 = W.reshape(1, d)
    b2 = b.reshape(1, 1)
    grid = (n // _BLK,)
    return pl.pallas_call(
        _fused_kernel,
        grid=grid,
        in_specs=[
            pl.BlockSpec((_BLK, d), lambda i: (i, 0)),
            pl.BlockSpec((1, d), lambda i: (0, 0)),
            pl.BlockSpec((1, 1), lambda i: (0, 0)),
        ],
        out_specs=pl.BlockSpec((_BLK, d), lambda i: (i, 0)),
        out_shape=jax.ShapeDtypeStruct((n, d), x.dtype),
    )(x, w2, b2)
